# Initial kernel scaffold; baseline (speedup 1.0000x reference)
import functools

import jax
import jax.numpy as jnp
from jax import lax
from jax.experimental import pallas as pl
from jax.experimental.pallas import tpu as pltpu
from jax.experimental.pallas import tpu_sc as plsc

N = 10000
E = 320000
D = 128
NP = 10240
NC = 2
NS = 16
NW = NC * NS
EPT = E // NW
CHUNK = 125
NCHUNK = EPT // CHUNK
AW = 144
ROWS_PER_TILE = NP // NS


def _mm_body(x_ref, w_ref, o_ref):
    o_ref[...] = jnp.dot(x_ref[...], w_ref[...],
                         preferred_element_type=jnp.float32)


def _tc_matmul(xp, wext):
    nb = NP // 1024
    return pl.pallas_call(
        _mm_body,
        grid=(nb,),
        in_specs=[
            pl.BlockSpec((1024, D), lambda i: (i, 0)),
            pl.BlockSpec((D, 256), lambda i: (0, 0)),
        ],
        out_specs=pl.BlockSpec((1024, 256), lambda i: (i, 0)),
        out_shape=jax.ShapeDtypeStruct((NP, 256), jnp.float32),
    )(xp, wext)


def _norm_mm_body(acc_ref, b_ref, w_ref, o_ref):
    r = acc_ref[0] + acc_ref[1]
    feat = r[:, :D]
    denom = r[:, D:D + 1]
    h = feat / (denom + 1e-16) + b_ref[...]
    h = jnp.maximum(h, 0.0)
    o_ref[...] = jnp.dot(h, w_ref[...], preferred_element_type=jnp.float32)


def _tc_norm_matmul(acc, b, wext):
    nb = NP // 1024
    return pl.pallas_call(
        _norm_mm_body,
        grid=(nb,),
        in_specs=[
            pl.BlockSpec((2, 1024, AW), lambda i: (0, i, 0)),
            pl.BlockSpec((1, D), lambda i: (0, 0)),
            pl.BlockSpec((D, 256), lambda i: (0, 0)),
        ],
        out_specs=pl.BlockSpec((1024, 256), lambda i: (i, 0)),
        out_shape=jax.ShapeDtypeStruct((NP, 256), jnp.float32),
    )(acc, b, wext)


def _final_body(acc_ref, b_ref, o_ref):
    r = acc_ref[0] + acc_ref[1]
    feat = r[:, :D]
    denom = r[:, D:D + 1]
    h = feat / (denom + 1e-16) + b_ref[...]
    m = jnp.max(h, axis=1, keepdims=True)
    lse = jnp.log(jnp.sum(jnp.exp(h - m), axis=1, keepdims=True))
    o_ref[...] = h - m - lse


def _tc_final(acc, b):
    nb = NP // 1024
    return pl.pallas_call(
        _final_body,
        grid=(nb,),
        in_specs=[
            pl.BlockSpec((2, 1024, AW), lambda i: (0, i, 0)),
            pl.BlockSpec((1, D), lambda i: (0, 0)),
        ],
        out_specs=pl.BlockSpec((1024, D), lambda i: (i, 0)),
        out_shape=jax.ShapeDtypeStruct((NP, D), jnp.float32),
    )(acc, b)


def _sc_edge_body(srcf_hbm, dstf_hbm, src2_hbm, dst2_hbm, asrc_hbm, adst_hbm,
                  h_hbm, acc_hbm,
                  asrc_v, adst_v, srcf_v, dstf_v, src2_v, dst2_v, ex_v,
                  hrows_v, rows_v, spm_acc, sem):
    c = lax.axis_index("c")
    s = lax.axis_index("s")
    w = c * NS + s

    pltpu.sync_copy(asrc_hbm, asrc_v)
    pltpu.sync_copy(adst_hbm, adst_v)
    pltpu.sync_copy(srcf_hbm.at[w], srcf_v)
    pltpu.sync_copy(dstf_hbm.at[w], dstf_v)
    pltpu.sync_copy(src2_hbm.at[w], src2_v)
    pltpu.sync_copy(dst2_hbm.at[w], dst2_v)

    def p1(k, _):
        sv = srcf_v[pl.ds(k * 16, 16)]
        dv = dstf_v[pl.ds(k * 16, 16)]
        e = plsc.load_gather(asrc_v, [sv]) + plsc.load_gather(adst_v, [dv])
        e = jnp.maximum(e, e * 0.2)
        ex_v[pl.ds(k * 16, 16)] = jnp.exp(e)
        return 0

    lax.fori_loop(0, EPT // 16, p1, 0)

    zero16 = jnp.zeros((16,), jnp.float32)

    def pz(i, _):
        rows_v[i // (AW // 16), pl.ds((i % (AW // 16)) * 16, 16)] = zero16
        return 0

    lax.fori_loop(0, CHUNK * (AW // 16), pz, 0)
    base = s * ROWS_PER_TILE
    for off in range(0, ROWS_PER_TILE, CHUNK):
        cnt = min(CHUNK, ROWS_PER_TILE - off)
        pltpu.sync_copy(rows_v.at[pl.ds(0, cnt)],
                        spm_acc.at[pl.ds(base + off, cnt)])
    plsc.subcore_barrier()

    onehot = jnp.where(lax.iota(jnp.int32, 16) == 0, 1.0, 0.0)

    def p2(k, _):
        pltpu.async_copy(h_hbm.at[src2_v.at[k]], hrows_v, sem).wait()

        def scale(i, _):
            av = plsc.load_gather(ex_v, [jnp.full((16,), k * CHUNK + i,
                                                  jnp.int32)])
            for j in range(D // 16):
                rows_v[i, pl.ds(j * 16, 16)] = (
                    hrows_v[i, pl.ds(j * 16, 16)] * av)
            rows_v[i, pl.ds(D, 16)] = av * onehot
            return 0

        lax.fori_loop(0, CHUNK, scale, 0)
        pltpu.sync_copy(rows_v, spm_acc.at[dst2_v.at[k]], add=True)
        return 0

    lax.fori_loop(0, NCHUNK, p2, 0)
    plsc.subcore_barrier()

    pltpu.sync_copy(spm_acc.at[pl.ds(base, ROWS_PER_TILE)],
                    acc_hbm.at[c, pl.ds(base, ROWS_PER_TILE)])


def _sc_edge(srcf, dstf, src2, dst2, asrc, adst, h):
    mesh = plsc.VectorSubcoreMesh(core_axis_name="c", subcore_axis_name="s")
    f = pl.kernel(
        _sc_edge_body,
        out_type=jax.ShapeDtypeStruct((NC, NP, AW), jnp.float32),
        mesh=mesh,
        scratch_types=[
            pltpu.VMEM((NP,), jnp.float32),
            pltpu.VMEM((NP,), jnp.float32),
            pltpu.VMEM((EPT,), jnp.int32),
            pltpu.VMEM((EPT,), jnp.int32),
            pltpu.VMEM((NCHUNK, CHUNK), jnp.int32),
            pltpu.VMEM((NCHUNK, CHUNK), jnp.int32),
            pltpu.VMEM((EPT,), jnp.float32),
            pltpu.VMEM((CHUNK, D), jnp.float32),
            pltpu.VMEM((CHUNK, AW), jnp.float32),
            pltpu.VMEM_SHARED((NP, AW), jnp.float32),
            pltpu.SemaphoreType.DMA,
        ],
    )
    return f(srcf, dstf, src2, dst2, asrc, adst, h)


def kernel(x, edge_index, W1, att_src1, att_dst1, b1,
           W2, att_src2, att_dst2, b2):
    src = edge_index[0].astype(jnp.int32)
    dst = edge_index[1].astype(jnp.int32)
    srcf = src.reshape(NW, EPT)
    dstf = dst.reshape(NW, EPT)
    src2 = src.reshape(NW, NCHUNK, CHUNK)
    dst2 = dst.reshape(NW, NCHUNK, CHUNK)

    def wext(W, a_s, a_d):
        return jnp.concatenate(
            [W, (W @ a_s)[:, None], (W @ a_d)[:, None],
             jnp.zeros((D, 256 - D - 2), jnp.float32)], axis=1)

    xp = jnp.pad(x, ((0, NP - N), (0, 0)))
    hext1 = _tc_matmul(xp, wext(W1, att_src1, att_dst1))
    acc1 = _sc_edge(srcf, dstf, src2, dst2,
                    hext1[:, D], hext1[:, D + 1], hext1[:, :D])
    hext2 = _tc_norm_matmul(acc1, b1[None, :], wext(W2, att_src2, att_dst2))
    acc2 = _sc_edge(srcf, dstf, src2, dst2,
                    hext2[:, D], hext2[:, D + 1], hext2[:, :D])
    out = _tc_final(acc2, b2[None, :])
    return out[:N]


# speedup vs baseline: 18.8117x; 18.8117x over previous
import jax
import jax.numpy as jnp
from jax import lax
from jax.experimental import pallas as pl
from jax.experimental.pallas import tpu as pltpu
from jax.experimental.pallas import tpu_sc as plsc

N = 10000
E = 320000
D = 128
NP = 10240
NC = 2
NS = 16
NW = NC * NS
EPT = E // NW
CHUNK = 80
NCHUNK = EPT // CHUNK
ROWS_PER_TILE = NP // NS


def _mm_body(x_ref, w_ref, o_ref):
    o_ref[...] = jnp.dot(x_ref[...], w_ref[...],
                         preferred_element_type=jnp.float32)


def _tc_matmul(xp, wext):
    nb = NP // 1024
    return pl.pallas_call(
        _mm_body,
        grid=(nb,),
        in_specs=[
            pl.BlockSpec((1024, D), lambda i: (i, 0)),
            pl.BlockSpec((D, 256), lambda i: (0, 0)),
        ],
        out_specs=pl.BlockSpec((1024, 256), lambda i: (i, 0)),
        out_shape=jax.ShapeDtypeStruct((NP, 256), jnp.float32),
    )(xp, wext)


def _norm_mm_body(acc_ref, dnm_ref, b_ref, w_ref, o_ref):
    feat = acc_ref[0] + acc_ref[1]
    denom = jnp.sum(dnm_ref[...], axis=0)[:, None]
    h = feat / (denom + 1e-16) + b_ref[...]
    h = jnp.maximum(h, 0.0)
    o_ref[...] = jnp.dot(h, w_ref[...], preferred_element_type=jnp.float32)


def _tc_norm_matmul(acc, dnm, b, wext):
    nb = NP // 1024
    return pl.pallas_call(
        _norm_mm_body,
        grid=(nb,),
        in_specs=[
            pl.BlockSpec((2, 1024, D), lambda i: (0, i, 0)),
            pl.BlockSpec((NW, 1024), lambda i: (0, i)),
            pl.BlockSpec((1, D), lambda i: (0, 0)),
            pl.BlockSpec((D, 256), lambda i: (0, 0)),
        ],
        out_specs=pl.BlockSpec((1024, 256), lambda i: (i, 0)),
        out_shape=jax.ShapeDtypeStruct((NP, 256), jnp.float32),
    )(acc, dnm, b, wext)


def _final_body(acc_ref, dnm_ref, b_ref, o_ref):
    feat = acc_ref[0] + acc_ref[1]
    denom = jnp.sum(dnm_ref[...], axis=0)[:, None]
    h = feat / (denom + 1e-16) + b_ref[...]
    m = jnp.max(h, axis=1, keepdims=True)
    lse = jnp.log(jnp.sum(jnp.exp(h - m), axis=1, keepdims=True))
    o_ref[...] = h - m - lse


def _tc_final(acc, dnm, b):
    nb = NP // 1024
    return pl.pallas_call(
        _final_body,
        grid=(nb,),
        in_specs=[
            pl.BlockSpec((2, 1024, D), lambda i: (0, i, 0)),
            pl.BlockSpec((NW, 1024), lambda i: (0, i)),
            pl.BlockSpec((1, D), lambda i: (0, 0)),
        ],
        out_specs=pl.BlockSpec((1024, D), lambda i: (i, 0)),
        out_shape=jax.ShapeDtypeStruct((NP, D), jnp.float32),
    )(acc, dnm, b)


def _sc_p1_body(src2_hbm, dst2_hbm, asrc_hbm, adst_hbm,
                ex_hbm, dnm_hbm,
                asrc_v, adst_v, src2_v, dst2_v, ex_v, dnm_v):
    c = lax.axis_index("c")
    s = lax.axis_index("s")
    w = c * NS + s

    pltpu.sync_copy(asrc_hbm, asrc_v)
    pltpu.sync_copy(adst_hbm, adst_v)
    pltpu.sync_copy(src2_hbm.at[w], src2_v)
    pltpu.sync_copy(dst2_hbm.at[w], dst2_v)

    zero16 = jnp.zeros((16,), jnp.float32)

    def z1(k, _):
        dnm_v[pl.ds(k * 16, 16)] = zero16
        return 0

    lax.fori_loop(0, NP // 16, z1, 0)

    def p1(k, _):
        for j in range(CHUNK // 16):
            sv = src2_v[k, pl.ds(j * 16, 16)]
            dv = dst2_v[k, pl.ds(j * 16, 16)]
            e = (plsc.load_gather(asrc_v, [sv])
                 + plsc.load_gather(adst_v, [dv]))
            e = jnp.maximum(e, e * 0.2)
            ex = jnp.exp(e)
            ex_v[k, pl.ds(j * 16, 16)] = ex
            plsc.addupdate_scatter(dnm_v, [dv], ex)
        return 0

    lax.fori_loop(0, NCHUNK, p1, 0)
    pltpu.sync_copy(ex_v, ex_hbm.at[w])
    pltpu.sync_copy(dnm_v, dnm_hbm.at[w])


def _sc_phase1(src2, dst2, asrc, adst):
    mesh = plsc.VectorSubcoreMesh(core_axis_name="c", subcore_axis_name="s")
    f = pl.kernel(
        _sc_p1_body,
        out_type=(jax.ShapeDtypeStruct((NW, NCHUNK, CHUNK), jnp.float32),
                  jax.ShapeDtypeStruct((NW, NP), jnp.float32)),
        mesh=mesh,
        compiler_params=pltpu.CompilerParams(needs_layout_passes=False),
        scratch_types=[
            pltpu.VMEM((NP,), jnp.float32),
            pltpu.VMEM((NP,), jnp.float32),
            pltpu.VMEM((NCHUNK, CHUNK), jnp.int32),
            pltpu.VMEM((NCHUNK, CHUNK), jnp.int32),
            pltpu.VMEM((NCHUNK, CHUNK), jnp.float32),
            pltpu.VMEM((NP,), jnp.float32),
        ],
    )
    return f(src2, dst2, asrc, adst)


def _sc_p2_body(src2_hbm, dst2_hbm, ex_hbm, h_hbm, acc_hbm,
                srcc_v, dstc_v, exc_v, rows_v, spm_acc, sem):
    c = lax.axis_index("c")
    s = lax.axis_index("s")
    w = c * NS + s

    zero16 = jnp.zeros((16,), jnp.float32)

    def pz(i, _):
        rows_v[i // (D // 16), pl.ds((i % (D // 16)) * 16, 16)] = zero16
        return 0

    lax.fori_loop(0, CHUNK * (D // 16), pz, 0)
    base = s * ROWS_PER_TILE
    for off in range(0, ROWS_PER_TILE, CHUNK):
        pltpu.sync_copy(rows_v, spm_acc.at[pl.ds(base + off, CHUNK)])
    plsc.subcore_barrier()

    def p2(k, _):
        pltpu.sync_copy(src2_hbm.at[w, k], srcc_v)
        pltpu.sync_copy(dst2_hbm.at[w, k], dstc_v)
        pltpu.sync_copy(ex_hbm.at[w, k], exc_v)
        pltpu.async_copy(h_hbm.at[srcc_v], rows_v, sem).wait()

        def scale(i, _):
            av = plsc.load_gather(exc_v, [jnp.full((16,), i, jnp.int32)])
            for j in range(D // 16):
                rows_v[i, pl.ds(j * 16, 16)] = (
                    rows_v[i, pl.ds(j * 16, 16)] * av)
            return 0

        lax.fori_loop(0, CHUNK, scale, 0)
        pltpu.sync_copy(rows_v, spm_acc.at[dstc_v], add=True)
        return 0

    lax.fori_loop(0, NCHUNK, p2, 0)
    plsc.subcore_barrier()

    pltpu.sync_copy(spm_acc.at[pl.ds(base, ROWS_PER_TILE)],
                    acc_hbm.at[c, pl.ds(base, ROWS_PER_TILE)])


def _sc_phase2(src2, dst2, ex2, h):
    mesh = plsc.VectorSubcoreMesh(core_axis_name="c", subcore_axis_name="s")
    f = pl.kernel(
        _sc_p2_body,
        out_type=jax.ShapeDtypeStruct((NC, NP, D), jnp.float32),
        mesh=mesh,
        compiler_params=pltpu.CompilerParams(needs_layout_passes=False),
        scratch_types=[
            pltpu.VMEM((CHUNK,), jnp.int32),
            pltpu.VMEM((CHUNK,), jnp.int32),
            pltpu.VMEM((CHUNK,), jnp.float32),
            pltpu.VMEM((CHUNK, D), jnp.float32),
            pltpu.VMEM_SHARED((NP, D), jnp.float32),
            pltpu.SemaphoreType.DMA,
        ],
    )
    return f(src2, dst2, ex2, h)


def kernel(x, edge_index, W1, att_src1, att_dst1, b1,
           W2, att_src2, att_dst2, b2):
    src = edge_index[0].astype(jnp.int32)
    dst = edge_index[1].astype(jnp.int32)
    src2 = src.reshape(NW, NCHUNK, CHUNK)
    dst2 = dst.reshape(NW, NCHUNK, CHUNK)

    def wext(W, a_s, a_d):
        return jnp.concatenate(
            [W, (W @ a_s)[:, None], (W @ a_d)[:, None],
             jnp.zeros((D, 256 - D - 2), jnp.float32)], axis=1)

    xp = jnp.pad(x, ((0, NP - N), (0, 0)))
    hext1 = _tc_matmul(xp, wext(W1, att_src1, att_dst1))
    ex1, dnm1 = _sc_phase1(src2, dst2, hext1[:, D], hext1[:, D + 1])
    acc1 = _sc_phase2(src2, dst2, ex1, hext1[:, :D])
    hext2 = _tc_norm_matmul(acc1, dnm1, b1[None, :],
                            wext(W2, att_src2, att_dst2))
    ex2, dnm2 = _sc_phase1(src2, dst2, hext2[:, D], hext2[:, D + 1])
    acc2 = _sc_phase2(src2, dst2, ex2, hext2[:, :D])
    out = _tc_final(acc2, dnm2, b2[None, :])
    return out[:N]
